# baseline (speedup 1.0000x reference)
import functools

import jax
import jax.numpy as jnp
from jax.experimental import pallas as pl
from jax.experimental.pallas import tpu as pltpu

TM = 256
NBUF = 5


def _body(w_hbm, a_hbm, f_hbm, wp_ref, bias_ref, o_ref,
          x_ref, f_ref, wbuf, abuf, wsem, asem, fsem, *, batch, out_f, n):
    nsteps = n // TM

    def tile_copies(tile, slot):
        return (
            pltpu.make_async_copy(
                w_hbm.at[pl.ds(tile * TM, TM), :], wbuf.at[slot], wsem.at[slot]),
            pltpu.make_async_copy(
                a_hbm.at[pl.ds(tile * TM, TM), :], abuf.at[slot], asem.at[slot]),
        )

    for s in range(NBUF):
        for c in tile_copies(s, s):
            c.start()

    fcopy = pltpu.make_async_copy(f_hbm, f_ref, fsem)
    fcopy.start()
    fcopy.wait()
    wp = wp_ref[...]
    for bi in range(batch):
        x_ref[:, bi * out_f:(bi + 1) * out_f] = jnp.dot(
            f_ref[bi], wp, preferred_element_type=jnp.float32)

    bias = bias_ref[...]

    def step(i, carry):
        s = jax.lax.rem(i, NBUF)
        for c in tile_copies(i, s):
            c.wait()
        aw = wbuf[s] * abuf[s]
        res = jnp.dot(aw, x_ref[...], preferred_element_type=jnp.float32)
        for bi in range(batch):
            o_ref[bi, pl.ds(i * TM, TM), :] = (
                res[:, bi * out_f:(bi + 1) * out_f] + bias)
        nxt = i + NBUF

        @pl.when(nxt < nsteps)
        def _():
            for c in tile_copies(nxt, s):
                c.start()

        return carry

    jax.lax.fori_loop(0, nsteps, step, 0)


@jax.jit
def kernel(weights, feats, adj, W, b):
    batch, n, in_f = feats.shape
    out_f = W.shape[1]

    hbm = pl.BlockSpec(memory_space=pltpu.MemorySpace.HBM)
    return pl.pallas_call(
        functools.partial(_body, batch=batch, out_f=out_f, n=n),
        in_specs=[
            hbm,
            hbm,
            hbm,
            pl.BlockSpec((in_f, out_f), None),
            pl.BlockSpec((1, out_f), None),
        ],
        out_specs=pl.BlockSpec((batch, n, out_f), None),
        out_shape=jax.ShapeDtypeStruct((batch, n, out_f), jnp.float32),
        scratch_shapes=[
            pltpu.VMEM((n, batch * out_f), jnp.float32),
            pltpu.VMEM((batch, n, in_f), jnp.float32),
            pltpu.VMEM((NBUF, TM, n), jnp.float32),
            pltpu.VMEM((NBUF, TM, n), jnp.float32),
            pltpu.SemaphoreType.DMA((NBUF,)),
            pltpu.SemaphoreType.DMA((NBUF,)),
            pltpu.SemaphoreType.DMA,
        ],
    )(weights, adj, feats, W, b)


# speedup vs baseline: 1.0083x; 1.0083x over previous
import functools

import jax
import jax.numpy as jnp
from jax.experimental import pallas as pl
from jax.experimental.pallas import tpu as pltpu

TM = 256
NBUF = 5


def _body(w_hbm, a_hbm, f_hbm, wp_ref, bias_ref, o_ref,
          x_ref, f_ref, wbuf, abuf, wsem, asem, fsem, *, batch, out_f, n):
    nsteps = n // TM

    def tile_copies(tile, slot):
        return (
            pltpu.make_async_copy(
                w_hbm.at[pl.ds(tile * TM, TM), :], wbuf.at[slot], wsem.at[slot]),
            pltpu.make_async_copy(
                a_hbm.at[pl.ds(tile * TM, TM), :], abuf.at[slot], asem.at[slot]),
        )

    for s in range(NBUF):
        for c in tile_copies(s, s):
            c.start()

    fcopy = pltpu.make_async_copy(f_hbm, f_ref, fsem)
    fcopy.start()
    fcopy.wait()
    wp = wp_ref[...]
    for bi in range(batch):
        x_ref[:, bi * out_f:(bi + 1) * out_f] = jnp.dot(
            f_ref[bi], wp, preferred_element_type=jnp.float32
        ).astype(jnp.bfloat16)

    bias = bias_ref[...]

    def step(i, carry):
        s = jax.lax.rem(i, NBUF)
        for c in tile_copies(i, s):
            c.wait()
        aw = (wbuf[s] * abuf[s]).astype(jnp.bfloat16)
        res = jnp.dot(aw, x_ref[...], preferred_element_type=jnp.float32)
        for bi in range(batch):
            o_ref[bi, pl.ds(i * TM, TM), :] = (
                res[:, bi * out_f:(bi + 1) * out_f] + bias)
        nxt = i + NBUF

        @pl.when(nxt < nsteps)
        def _():
            for c in tile_copies(nxt, s):
                c.start()

        return carry

    jax.lax.fori_loop(0, nsteps, step, 0)


@jax.jit
def kernel(weights, feats, adj, W, b):
    batch, n, in_f = feats.shape
    out_f = W.shape[1]

    hbm = pl.BlockSpec(memory_space=pltpu.MemorySpace.HBM)
    return pl.pallas_call(
        functools.partial(_body, batch=batch, out_f=out_f, n=n),
        in_specs=[
            hbm,
            hbm,
            hbm,
            pl.BlockSpec((in_f, out_f), None),
            pl.BlockSpec((1, out_f), None),
        ],
        out_specs=pl.BlockSpec((batch, n, out_f), None),
        out_shape=jax.ShapeDtypeStruct((batch, n, out_f), jnp.float32),
        scratch_shapes=[
            pltpu.VMEM((n, batch * out_f), jnp.bfloat16),
            pltpu.VMEM((batch, n, in_f), jnp.float32),
            pltpu.VMEM((NBUF, TM, n), jnp.float32),
            pltpu.VMEM((NBUF, TM, n), jnp.float32),
            pltpu.SemaphoreType.DMA((NBUF,)),
            pltpu.SemaphoreType.DMA((NBUF,)),
            pltpu.SemaphoreType.DMA,
        ],
    )(weights, adj, feats, W, b)


# speedup vs baseline: 1.0633x; 1.0545x over previous
import functools

import jax
import jax.numpy as jnp
from jax.experimental import pallas as pl
from jax.experimental.pallas import tpu as pltpu

TM = 256
NBUF = 5


def _body(w_hbm, a_hbm, f_ref, wp_ref, bias_ref, o_ref,
          x_ref, wbuf, abuf, wsem, asem, *, batch, out_f, n):
    nsteps = n // TM

    def tile_copies(tile, slot):
        return (
            pltpu.make_async_copy(
                w_hbm.at[pl.ds(tile * TM, TM), :], wbuf.at[slot], wsem.at[slot]),
            pltpu.make_async_copy(
                a_hbm.at[pl.ds(tile * TM, TM), :], abuf.at[slot], asem.at[slot]),
        )

    for s in range(NBUF):
        for c in tile_copies(s, s):
            c.start()

    wp = wp_ref[...]
    for bi in range(batch):
        x_ref[:, bi * out_f:(bi + 1) * out_f] = jnp.dot(
            f_ref[bi], wp, preferred_element_type=jnp.float32
        ).astype(jnp.bfloat16)

    bias = bias_ref[...]

    def step(i, carry):
        s = jax.lax.rem(i, NBUF)
        for c in tile_copies(i, s):
            c.wait()
        aw = (wbuf[s] * abuf[s]).astype(jnp.bfloat16)
        res = jnp.dot(aw, x_ref[...], preferred_element_type=jnp.float32)
        for bi in range(batch):
            o_ref[bi, pl.ds(i * TM, TM), :] = (
                res[:, bi * out_f:(bi + 1) * out_f] + bias)
        nxt = i + NBUF

        @pl.when(nxt < nsteps)
        def _():
            for c in tile_copies(nxt, s):
                c.start()

        return carry

    jax.lax.fori_loop(0, nsteps, step, 0)


@jax.jit
def kernel(weights, feats, adj, W, b):
    batch, n, in_f = feats.shape
    out_f = W.shape[1]

    hbm = pl.BlockSpec(memory_space=pltpu.MemorySpace.HBM)
    return pl.pallas_call(
        functools.partial(_body, batch=batch, out_f=out_f, n=n),
        in_specs=[
            hbm,
            hbm,
            pl.BlockSpec((batch, n, in_f), None),
            pl.BlockSpec((in_f, out_f), None),
            pl.BlockSpec((1, out_f), None),
        ],
        out_specs=pl.BlockSpec((batch, n, out_f), None),
        out_shape=jax.ShapeDtypeStruct((batch, n, out_f), jnp.float32),
        scratch_shapes=[
            pltpu.VMEM((n, batch * out_f), jnp.bfloat16),
            pltpu.VMEM((NBUF, TM, n), jnp.float32),
            pltpu.VMEM((NBUF, TM, n), jnp.float32),
            pltpu.SemaphoreType.DMA((NBUF,)),
            pltpu.SemaphoreType.DMA((NBUF,)),
        ],
    )(weights, adj, feats, W, b)


# speedup vs baseline: 1.1517x; 1.0832x over previous
import functools

import jax
import jax.numpy as jnp
from jax.experimental import pallas as pl
from jax.experimental.pallas import tpu as pltpu

TM = 256
NBUF = 5


def _body(w_hbm, a_hbm, f_ref, wp_ref, bias_ref, o_ref,
          x_ref, wbuf, abuf, wsem, asem, *, batch, out_f, n):
    nsteps = n // TM

    def tile_copies(tile, slot):
        return (
            pltpu.make_async_copy(
                w_hbm.at[pl.ds(tile * TM, TM), :], wbuf.at[slot], wsem.at[slot]),
            pltpu.make_async_copy(
                a_hbm.at[pl.ds(tile * TM, TM), :], abuf.at[slot], asem.at[slot]),
        )

    for s in range(NBUF):
        for c in tile_copies(s, s):
            c.start()

    wp = wp_ref[...]
    for bi in range(batch):
        x_ref[:, bi * out_f:(bi + 1) * out_f] = jnp.dot(
            f_ref[bi], wp, preferred_element_type=jnp.float32
        ).astype(jnp.bfloat16)

    bias = jnp.tile(bias_ref[...], (1, batch))

    def step(i, carry):
        s = jax.lax.rem(i, NBUF)
        for c in tile_copies(i, s):
            c.wait()
        aw = (wbuf[s] * abuf[s]).astype(jnp.bfloat16)
        res = jnp.dot(aw, x_ref[...], preferred_element_type=jnp.float32)
        o_ref[pl.ds(i * TM, TM), :] = res + bias
        nxt = i + NBUF

        @pl.when(nxt < nsteps)
        def _():
            for c in tile_copies(nxt, s):
                c.start()

        return carry

    jax.lax.fori_loop(0, nsteps, step, 0)


@jax.jit
def kernel(weights, feats, adj, W, b):
    batch, n, in_f = feats.shape
    out_f = W.shape[1]

    hbm = pl.BlockSpec(memory_space=pltpu.MemorySpace.HBM)
    out = pl.pallas_call(
        functools.partial(_body, batch=batch, out_f=out_f, n=n),
        in_specs=[
            hbm,
            hbm,
            pl.BlockSpec((batch, n, in_f), None),
            pl.BlockSpec((in_f, out_f), None),
            pl.BlockSpec((1, out_f), None),
        ],
        out_specs=pl.BlockSpec((n, batch * out_f), None),
        out_shape=jax.ShapeDtypeStruct((n, batch * out_f), jnp.float32),
        scratch_shapes=[
            pltpu.VMEM((n, batch * out_f), jnp.bfloat16),
            pltpu.VMEM((NBUF, TM, n), jnp.float32),
            pltpu.VMEM((NBUF, TM, n), jnp.float32),
            pltpu.SemaphoreType.DMA((NBUF,)),
            pltpu.SemaphoreType.DMA((NBUF,)),
        ],
    )(weights, adj, feats, W, b)
    return out.reshape(n, batch, out_f).transpose(1, 0, 2)


# speedup vs baseline: 1.1689x; 1.0149x over previous
import functools

import jax
import jax.numpy as jnp
from jax.experimental import pallas as pl
from jax.experimental.pallas import tpu as pltpu

TM = 256
NBUF = 5


def _body(w_hbm, a_hbm, f_ref, wp_ref, bias_ref, o_ref,
          x_ref, wbuf, abuf, wsem, asem, *, batch, out_f, n):
    nsteps = n // TM

    half = n // 2

    def tile_copies(tile, slot):
        cs = []
        for src, buf, sem in ((w_hbm, wbuf, wsem), (a_hbm, abuf, asem)):
            for h in range(2):
                cs.append(pltpu.make_async_copy(
                    src.at[pl.ds(tile * TM, TM), pl.ds(h * half, half)],
                    buf.at[slot, :, pl.ds(h * half, half)],
                    sem.at[slot]))
        return cs

    for s in range(NBUF):
        for c in tile_copies(s, s):
            c.start()

    wp = wp_ref[...]
    for bi in range(batch):
        x_ref[:, bi * out_f:(bi + 1) * out_f] = jnp.dot(
            f_ref[bi], wp, preferred_element_type=jnp.float32
        ).astype(jnp.bfloat16)

    bias = jnp.tile(bias_ref[...], (1, batch))

    def step(i, carry):
        s = jax.lax.rem(i, NBUF)
        for c in tile_copies(i, s):
            c.wait()
        aw = (wbuf[s] * abuf[s]).astype(jnp.bfloat16)
        res = jnp.dot(aw, x_ref[...], preferred_element_type=jnp.float32)
        o_ref[pl.ds(i * TM, TM), :] = res + bias
        nxt = i + NBUF

        @pl.when(nxt < nsteps)
        def _():
            for c in tile_copies(nxt, s):
                c.start()

        return carry

    jax.lax.fori_loop(0, nsteps, step, 0)


@jax.jit
def kernel(weights, feats, adj, W, b):
    batch, n, in_f = feats.shape
    out_f = W.shape[1]

    hbm = pl.BlockSpec(memory_space=pltpu.MemorySpace.HBM)
    out = pl.pallas_call(
        functools.partial(_body, batch=batch, out_f=out_f, n=n),
        in_specs=[
            hbm,
            hbm,
            pl.BlockSpec((batch, n, in_f), None),
            pl.BlockSpec((in_f, out_f), None),
            pl.BlockSpec((1, out_f), None),
        ],
        out_specs=pl.BlockSpec((n, batch * out_f), None),
        out_shape=jax.ShapeDtypeStruct((n, batch * out_f), jnp.float32),
        scratch_shapes=[
            pltpu.VMEM((n, batch * out_f), jnp.bfloat16),
            pltpu.VMEM((NBUF, TM, n), jnp.float32),
            pltpu.VMEM((NBUF, TM, n), jnp.float32),
            pltpu.SemaphoreType.DMA((NBUF,)),
            pltpu.SemaphoreType.DMA((NBUF,)),
        ],
    )(weights, adj, feats, W, b)
    return out.reshape(n, batch, out_f).transpose(1, 0, 2)
